# baseline (speedup 1.0000x reference)
import jax
import jax.numpy as jnp
from jax import lax
from jax.experimental import pallas as pl
from jax.experimental.pallas import tpu as pltpu

_D, _C, _H, _B = 32, 8, 128, 1024


def _onehot_rows(idx_row):
    return (lax.broadcasted_iota(jnp.int32, (_C, _B), 0) == idx_row).astype(
        jnp.float32)


def _gibbs_body(xT_ref, w1r_ref, w1t_ref, b1_ref, w2b_ref, b2_ref, g_ref,
                out_ref, s_ref):
    rep = jnp.broadcast_to(
        xT_ref[...].reshape(_D, 1, _B), (_D, _C, _B)).reshape(_D * _C, _B)
    cmod = lax.rem(lax.broadcasted_iota(jnp.int32, (_D * _C, _B), 0), _C)
    oht = (rep == cmod).astype(jnp.float32)
    s0 = lax.dot_general(w1t_ref[...], oht, (((1,), (0,)), ((), ())),
                         preferred_element_type=jnp.float32)
    s_ref[...] = s0 + b1_ref[...]

    def prep(i):
        x_i = xT_ref[i].reshape(1, _B)
        ohx = _onehot_rows(x_i)
        w_i = w1r_ref[i]
        sub = lax.dot_general(w_i, ohx, (((0,), (0,)), ((), ())),
                              preferred_element_type=jnp.float32)
        return sub

    def step(i, sub):
        w_i = w1r_ref[i]
        base = s_ref[...] - sub
        w_iT = lax.dot_general(w_i, jnp.eye(_C, dtype=jnp.float32),
                               (((0,), (0,)), ((), ())),
                               preferred_element_type=jnp.float32)
        t2 = jnp.concatenate(
            [jnp.maximum(base + w_iT[:, c:c + 1], 0.0) for c in range(_C)],
            axis=0)
        scores = lax.dot_general(w2b_ref[...], t2, (((1,), (0,)), ((), ())),
                                 preferred_element_type=jnp.float32)
        g_i = g_ref[pl.ds(i * _C, _C), :]
        v = g_i - (scores + b2_ref[...])
        maxv = jnp.max(v, axis=0, keepdims=True)
        iota_c = lax.broadcasted_iota(jnp.int32, (_C, _B), 0)
        best_c = jnp.min(jnp.where(v == maxv, iota_c, _C),
                         axis=0, keepdims=True)
        ohc = _onehot_rows(best_c)
        addw = lax.dot_general(w_i, ohc, (((0,), (0,)), ((), ())),
                               preferred_element_type=jnp.float32)
        s_ref[...] = base + addw
        out_ref[pl.ds(i, 1), :] = best_c
        return prep(jnp.minimum(i + 1, _D - 1))

    lax.fori_loop(0, _D, step, prep(0))


def _gumbel_table(num_rounds):
    base_key = jax.random.key(42)
    steps = jnp.arange(_D) * num_rounds
    keys = jax.vmap(lambda s: jax.random.fold_in(base_key, s))(steps)
    g = jax.vmap(lambda k: jax.random.gumbel(k, (_B, _C), jnp.float32))(keys)
    return jnp.swapaxes(g, 1, 2).reshape(_D * _C, _B)


def kernel(init_samples, num_rounds, W1, b1, w2, b2):
    xT = init_samples.T.astype(jnp.int32)
    w1r = W1.reshape(_D, _C, _H)
    w1t = W1.T
    w2b = (jnp.eye(_C, dtype=jnp.float32)[:, :, None]
           * w2[None, None, :]).reshape(_C, _C * _H)
    gT = _gumbel_table(num_rounds)
    outT = pl.pallas_call(
        _gibbs_body,
        out_shape=jax.ShapeDtypeStruct((_D, _B), jnp.int32),
        scratch_shapes=[pltpu.VMEM((_H, _B), jnp.float32)],
    )(xT, w1r, w1t, b1.reshape(_H, 1), w2b, b2.reshape(1, 1), gT)
    return outT.T.astype(init_samples.dtype)


# speedup vs baseline: 1.0308x; 1.0308x over previous
import jax
import jax.numpy as jnp
from jax import lax
from jax.experimental import pallas as pl
from jax.experimental.pallas import tpu as pltpu

_D, _C, _H, _B = 32, 8, 128, 1024


def _onehot_rows(idx_row):
    return (lax.broadcasted_iota(jnp.int32, (_C, _B), 0) == idx_row).astype(
        jnp.float32)


def _gibbs_body(xT_ref, w1r_ref, w1t_ref, b1_ref, w2b_ref, b2_ref, g_ref,
                out_ref, s_ref):
    rep = jnp.broadcast_to(
        xT_ref[...].reshape(_D, 1, _B), (_D, _C, _B)).reshape(_D * _C, _B)
    cmod = lax.rem(lax.broadcasted_iota(jnp.int32, (_D * _C, _B), 0), _C)
    oht = (rep == cmod).astype(jnp.float32)
    s0 = lax.dot_general(w1t_ref[...], oht, (((1,), (0,)), ((), ())),
                         preferred_element_type=jnp.float32)
    s_ref[...] = s0 + b1_ref[...]

    def step(i, carry):
        x_i = xT_ref[i].reshape(1, _B)
        ohx = _onehot_rows(x_i)
        w_i = w1r_ref[i]
        sub = lax.dot_general(w_i, ohx, (((0,), (0,)), ((), ())),
                              preferred_element_type=jnp.float32)
        base = s_ref[...] - sub
        w_iT = lax.dot_general(w_i, jnp.eye(_C, dtype=jnp.float32),
                               (((0,), (0,)), ((), ())),
                               preferred_element_type=jnp.float32)
        t2 = jnp.concatenate(
            [jnp.maximum(base + w_iT[:, c:c + 1], 0.0) for c in range(_C)],
            axis=0)
        scores = lax.dot_general(w2b_ref[...], t2, (((1,), (0,)), ((), ())),
                                 preferred_element_type=jnp.float32)
        g_i = g_ref[pl.ds(i * _C, _C), :]
        v = g_i - (scores + b2_ref[...])
        maxv = jnp.max(v, axis=0, keepdims=True)
        iota_c = lax.broadcasted_iota(jnp.int32, (_C, _B), 0)
        best_c = jnp.min(jnp.where(v == maxv, iota_c, _C),
                         axis=0, keepdims=True)
        ohc = _onehot_rows(best_c)
        addw = lax.dot_general(w_i, ohc, (((0,), (0,)), ((), ())),
                               preferred_element_type=jnp.float32)
        s_ref[...] = base + addw
        out_ref[pl.ds(i, 1), :] = best_c
        return carry

    lax.fori_loop(0, _D, step, 0)


def _gumbel_table(num_rounds):
    base_key = jax.random.key(42)
    steps = jnp.arange(_D) * num_rounds
    keys = jax.vmap(lambda s: jax.random.fold_in(base_key, s))(steps)
    g = jax.vmap(lambda k: jax.random.gumbel(k, (_B, _C), jnp.float32))(keys)
    return jnp.swapaxes(g, 1, 2).reshape(_D * _C, _B)


def kernel(init_samples, num_rounds, W1, b1, w2, b2):
    xT = init_samples.T.astype(jnp.int32)
    w1r = W1.reshape(_D, _C, _H)
    w1t = W1.T
    w2b = (jnp.eye(_C, dtype=jnp.float32)[:, :, None]
           * w2[None, None, :]).reshape(_C, _C * _H)
    gT = _gumbel_table(num_rounds)
    outT = pl.pallas_call(
        _gibbs_body,
        out_shape=jax.ShapeDtypeStruct((_D, _B), jnp.int32),
        scratch_shapes=[pltpu.VMEM((_H, _B), jnp.float32)],
    )(xT, w1r, w1t, b1.reshape(_H, 1), w2b, b2.reshape(1, 1), gT)
    return outT.T.astype(init_samples.dtype)


# speedup vs baseline: 1.0799x; 1.0476x over previous
import jax
import jax.numpy as jnp
from jax import lax
from jax.experimental import pallas as pl
from jax.experimental.pallas import tpu as pltpu

_D, _C, _H, _B = 32, 8, 128, 1024


def _onehot_rows(idx_row):
    return (lax.broadcasted_iota(jnp.int32, (_C, _B), 0) == idx_row).astype(
        jnp.float32)


def _gibbs_body(xT_ref, w1r_ref, w1t_ref, b1_ref, w2b_ref, b2_ref, g_ref,
                out_ref, s_ref):
    rep = jnp.broadcast_to(
        xT_ref[...].reshape(_D, 1, _B), (_D, _C, _B)).reshape(_D * _C, _B)
    cmod = lax.rem(lax.broadcasted_iota(jnp.int32, (_D * _C, _B), 0), _C)
    oht = (rep == cmod).astype(jnp.float32)
    s0 = lax.dot_general(w1t_ref[...], oht, (((1,), (0,)), ((), ())),
                         preferred_element_type=jnp.float32)
    s_ref[...] = s0 + b1_ref[...]

    def step(i, carry):
        x_i = xT_ref[i].reshape(1, _B)
        ohx = _onehot_rows(x_i)
        w_i = w1r_ref[i]
        sub = lax.dot_general(w_i, ohx, (((0,), (0,)), ((), ())),
                              preferred_element_type=jnp.float32)
        base = s_ref[...] - sub
        w_iT = lax.dot_general(w_i, jnp.eye(_C, dtype=jnp.float32),
                               (((0,), (0,)), ((), ())),
                               preferred_element_type=jnp.float32)
        t2 = jnp.concatenate(
            [jnp.maximum(base + w_iT[:, c:c + 1], 0.0) for c in range(_C)],
            axis=0)
        scores = lax.dot_general(w2b_ref[...], t2, (((1,), (0,)), ((), ())),
                                 preferred_element_type=jnp.float32)
        g_i = g_ref[pl.ds(i * _C, _C), :]
        v = g_i - (scores + b2_ref[...])
        maxv = jnp.max(v, axis=0, keepdims=True)
        iota_c = lax.broadcasted_iota(jnp.int32, (_C, _B), 0)
        best_c = jnp.min(jnp.where(v == maxv, iota_c, _C),
                         axis=0, keepdims=True)
        ohc = _onehot_rows(best_c)
        addw = lax.dot_general(w_i, ohc, (((0,), (0,)), ((), ())),
                               preferred_element_type=jnp.float32)
        s_ref[...] = base + addw
        out_ref[pl.ds(i, 1), :] = best_c
        return carry

    def step2(j, carry):
        return step(2 * j + 1, step(2 * j, carry))

    lax.fori_loop(0, _D // 2, step2, 0)


def _gumbel_table(num_rounds):
    base_key = jax.random.key(42)
    steps = jnp.arange(_D) * num_rounds
    keys = jax.vmap(lambda s: jax.random.fold_in(base_key, s))(steps)
    g = jax.vmap(lambda k: jax.random.gumbel(k, (_B, _C), jnp.float32))(keys)
    return jnp.swapaxes(g, 1, 2).reshape(_D * _C, _B)


def kernel(init_samples, num_rounds, W1, b1, w2, b2):
    xT = init_samples.T.astype(jnp.int32)
    w1r = W1.reshape(_D, _C, _H)
    w1t = W1.T
    w2b = (jnp.eye(_C, dtype=jnp.float32)[:, :, None]
           * w2[None, None, :]).reshape(_C, _C * _H)
    gT = _gumbel_table(num_rounds)
    outT = pl.pallas_call(
        _gibbs_body,
        out_shape=jax.ShapeDtypeStruct((_D, _B), jnp.int32),
        scratch_shapes=[pltpu.VMEM((_H, _B), jnp.float32)],
    )(xT, w1r, w1t, b1.reshape(_H, 1), w2b, b2.reshape(1, 1), gT)
    return outT.T.astype(init_samples.dtype)


# speedup vs baseline: 1.1173x; 1.0346x over previous
import jax
import jax.numpy as jnp
from jax import lax
from jax.experimental import pallas as pl
from jax.experimental.pallas import tpu as pltpu

_D, _C, _H, _B = 32, 8, 128, 1024


def _onehot_rows(idx_row):
    return (lax.broadcasted_iota(jnp.int32, (_C, _B), 0) == idx_row).astype(
        jnp.float32)


def _gibbs_body(xT_ref, w1r_ref, w1t_ref, b1_ref, w2b_ref, b2_ref, g_ref,
                out_ref, s_ref):
    rep = jnp.broadcast_to(
        xT_ref[...].reshape(_D, 1, _B), (_D, _C, _B)).reshape(_D * _C, _B)
    cmod = lax.rem(lax.broadcasted_iota(jnp.int32, (_D * _C, _B), 0), _C)
    oht = (rep == cmod).astype(jnp.float32)
    s0 = lax.dot_general(w1t_ref[...], oht, (((1,), (0,)), ((), ())),
                         preferred_element_type=jnp.float32)
    s_ref[...] = s0 + b1_ref[...]

    def step(i, carry):
        x_i = xT_ref[i].reshape(1, _B)
        ohx = _onehot_rows(x_i)
        w_i = w1r_ref[i]
        sub = lax.dot_general(w_i, ohx, (((0,), (0,)), ((), ())),
                              preferred_element_type=jnp.float32)
        base = s_ref[...] - sub
        w_iT = lax.dot_general(w_i, jnp.eye(_C, dtype=jnp.float32),
                               (((0,), (0,)), ((), ())),
                               preferred_element_type=jnp.float32)
        t2 = jnp.concatenate(
            [jnp.maximum(base + w_iT[:, c:c + 1], 0.0) for c in range(_C)],
            axis=0)
        scores = lax.dot_general(w2b_ref[...], t2, (((1,), (0,)), ((), ())),
                                 preferred_element_type=jnp.float32)
        g_i = g_ref[pl.ds(i * _C, _C), :]
        v = g_i - (scores + b2_ref[...])
        maxv = jnp.max(v, axis=0, keepdims=True)
        iota_c = lax.broadcasted_iota(jnp.int32, (_C, _B), 0)
        best_c = jnp.min(jnp.where(v == maxv, iota_c, _C),
                         axis=0, keepdims=True)
        ohc = _onehot_rows(best_c)
        addw = lax.dot_general(w_i, ohc, (((0,), (0,)), ((), ())),
                               preferred_element_type=jnp.float32)
        s_ref[...] = base + addw
        out_ref[pl.ds(i, 1), :] = best_c
        return carry

    def step4(j, carry):
        for k in range(4):
            carry = step(4 * j + k, carry)
        return carry

    lax.fori_loop(0, _D // 4, step4, 0)


def _gumbel_table(num_rounds):
    base_key = jax.random.key(42)
    steps = jnp.arange(_D) * num_rounds
    keys = jax.vmap(lambda s: jax.random.fold_in(base_key, s))(steps)
    g = jax.vmap(lambda k: jax.random.gumbel(k, (_B, _C), jnp.float32))(keys)
    return jnp.swapaxes(g, 1, 2).reshape(_D * _C, _B)


def kernel(init_samples, num_rounds, W1, b1, w2, b2):
    xT = init_samples.T.astype(jnp.int32)
    w1r = W1.reshape(_D, _C, _H)
    w1t = W1.T
    w2b = (jnp.eye(_C, dtype=jnp.float32)[:, :, None]
           * w2[None, None, :]).reshape(_C, _C * _H)
    gT = _gumbel_table(num_rounds)
    outT = pl.pallas_call(
        _gibbs_body,
        out_shape=jax.ShapeDtypeStruct((_D, _B), jnp.int32),
        scratch_shapes=[pltpu.VMEM((_H, _B), jnp.float32)],
    )(xT, w1r, w1t, b1.reshape(_H, 1), w2b, b2.reshape(1, 1), gT)
    return outT.T.astype(init_samples.dtype)


# speedup vs baseline: 1.1386x; 1.0191x over previous
import jax
import jax.numpy as jnp
from jax import lax
from jax.experimental import pallas as pl
from jax.experimental.pallas import tpu as pltpu

_D, _C, _H, _B = 32, 8, 128, 1024


def _onehot_rows(idx_row):
    return (lax.broadcasted_iota(jnp.int32, (_C, _B), 0) == idx_row).astype(
        jnp.float32)


def _gibbs_body(xT_ref, w1r_ref, w1t_ref, b1_ref, w2b_ref, b2_ref, g_ref,
                out_ref, s_ref):
    rep = jnp.broadcast_to(
        xT_ref[...].reshape(_D, 1, _B), (_D, _C, _B)).reshape(_D * _C, _B)
    cmod = lax.rem(lax.broadcasted_iota(jnp.int32, (_D * _C, _B), 0), _C)
    oht = (rep == cmod).astype(jnp.float32)
    s0 = lax.dot_general(w1t_ref[...], oht, (((1,), (0,)), ((), ())),
                         preferred_element_type=jnp.float32)
    s_ref[...] = s0 + b1_ref[...]

    def step(i, carry):
        x_i = xT_ref[i].reshape(1, _B)
        ohx = _onehot_rows(x_i)
        w_i = w1r_ref[i]
        sub = lax.dot_general(w_i, ohx, (((0,), (0,)), ((), ())),
                              preferred_element_type=jnp.float32)
        base = s_ref[...] - sub
        w_iT = lax.dot_general(w_i, jnp.eye(_C, dtype=jnp.float32),
                               (((0,), (0,)), ((), ())),
                               preferred_element_type=jnp.float32)
        t2 = jnp.concatenate(
            [jnp.maximum(base + w_iT[:, c:c + 1], 0.0) for c in range(_C)],
            axis=0)
        scores = lax.dot_general(w2b_ref[...], t2, (((1,), (0,)), ((), ())),
                                 preferred_element_type=jnp.float32)
        g_i = g_ref[pl.ds(i * _C, _C), :]
        v = g_i - (scores + b2_ref[...])
        maxv = jnp.max(v, axis=0, keepdims=True)
        iota_c = lax.broadcasted_iota(jnp.int32, (_C, _B), 0)
        best_c = jnp.min(jnp.where(v == maxv, iota_c, _C),
                         axis=0, keepdims=True)
        ohc = _onehot_rows(best_c)
        addw = lax.dot_general(w_i, ohc, (((0,), (0,)), ((), ())),
                               preferred_element_type=jnp.float32)
        s_ref[...] = base + addw
        out_ref[pl.ds(i, 1), :] = best_c
        return carry

    def step8(j, carry):
        for k in range(8):
            carry = step(8 * j + k, carry)
        return carry

    lax.fori_loop(0, _D // 8, step8, 0)


def _gumbel_table(num_rounds):
    base_key = jax.random.key(42)
    steps = jnp.arange(_D) * num_rounds
    keys = jax.vmap(lambda s: jax.random.fold_in(base_key, s))(steps)
    g = jax.vmap(lambda k: jax.random.gumbel(k, (_B, _C), jnp.float32))(keys)
    return jnp.swapaxes(g, 1, 2).reshape(_D * _C, _B)


def kernel(init_samples, num_rounds, W1, b1, w2, b2):
    xT = init_samples.T.astype(jnp.int32)
    w1r = W1.reshape(_D, _C, _H)
    w1t = W1.T
    w2b = (jnp.eye(_C, dtype=jnp.float32)[:, :, None]
           * w2[None, None, :]).reshape(_C, _C * _H)
    gT = _gumbel_table(num_rounds)
    outT = pl.pallas_call(
        _gibbs_body,
        out_shape=jax.ShapeDtypeStruct((_D, _B), jnp.int32),
        scratch_shapes=[pltpu.VMEM((_H, _B), jnp.float32)],
    )(xT, w1r, w1t, b1.reshape(_H, 1), w2b, b2.reshape(1, 1), gT)
    return outT.T.astype(init_samples.dtype)


# speedup vs baseline: 1.1543x; 1.0138x over previous
import jax
import jax.numpy as jnp
from jax import lax
from jax.experimental import pallas as pl
from jax.experimental.pallas import tpu as pltpu

_D, _C, _H, _B = 32, 8, 128, 1024


def _onehot_rows(idx_row):
    return (lax.broadcasted_iota(jnp.int32, (_C, _B), 0) == idx_row).astype(
        jnp.float32)


def _gibbs_body(xT_ref, w1r_ref, w1t_ref, b1_ref, w2b_ref, b2_ref, g_ref,
                out_ref, s_ref):
    rep = jnp.broadcast_to(
        xT_ref[...].reshape(_D, 1, _B), (_D, _C, _B)).reshape(_D * _C, _B)
    cmod = lax.rem(lax.broadcasted_iota(jnp.int32, (_D * _C, _B), 0), _C)
    oht = (rep == cmod).astype(jnp.float32)
    s0 = lax.dot_general(w1t_ref[...], oht, (((1,), (0,)), ((), ())),
                         preferred_element_type=jnp.float32)
    s_ref[...] = s0 + b1_ref[...]

    def step(i, carry):
        x_i = xT_ref[i].reshape(1, _B)
        ohx = _onehot_rows(x_i)
        w_i = w1r_ref[i]
        sub = lax.dot_general(w_i, ohx, (((0,), (0,)), ((), ())),
                              preferred_element_type=jnp.float32)
        base = s_ref[...] - sub
        w_iT = lax.dot_general(w_i, jnp.eye(_C, dtype=jnp.float32),
                               (((0,), (0,)), ((), ())),
                               preferred_element_type=jnp.float32)
        t2 = jnp.concatenate(
            [jnp.maximum(base + w_iT[:, c:c + 1], 0.0) for c in range(_C)],
            axis=0)
        scores = lax.dot_general(w2b_ref[...], t2, (((1,), (0,)), ((), ())),
                                 preferred_element_type=jnp.float32)
        g_i = g_ref[pl.ds(i * _C, _C), :]
        v = g_i - (scores + b2_ref[...])
        maxv = jnp.max(v, axis=0, keepdims=True)
        iota_c = lax.broadcasted_iota(jnp.int32, (_C, _B), 0)
        best_c = jnp.min(jnp.where(v == maxv, iota_c, _C),
                         axis=0, keepdims=True)
        ohc = _onehot_rows(best_c)
        addw = lax.dot_general(w_i, ohc, (((0,), (0,)), ((), ())),
                               preferred_element_type=jnp.float32)
        s_ref[...] = base + addw
        out_ref[pl.ds(i, 1), :] = best_c
        return carry

    for i in range(_D):
        step(i, 0)


def _gumbel_table(num_rounds):
    base_key = jax.random.key(42)
    steps = jnp.arange(_D) * num_rounds
    keys = jax.vmap(lambda s: jax.random.fold_in(base_key, s))(steps)
    g = jax.vmap(lambda k: jax.random.gumbel(k, (_B, _C), jnp.float32))(keys)
    return jnp.swapaxes(g, 1, 2).reshape(_D * _C, _B)


def kernel(init_samples, num_rounds, W1, b1, w2, b2):
    xT = init_samples.T.astype(jnp.int32)
    w1r = W1.reshape(_D, _C, _H)
    w1t = W1.T
    w2b = (jnp.eye(_C, dtype=jnp.float32)[:, :, None]
           * w2[None, None, :]).reshape(_C, _C * _H)
    gT = _gumbel_table(num_rounds)
    outT = pl.pallas_call(
        _gibbs_body,
        out_shape=jax.ShapeDtypeStruct((_D, _B), jnp.int32),
        scratch_shapes=[pltpu.VMEM((_H, _B), jnp.float32)],
    )(xT, w1r, w1t, b1.reshape(_H, 1), w2b, b2.reshape(1, 1), gT)
    return outT.T.astype(init_samples.dtype)
